# baseline (speedup 1.0000x reference)
import jax
import jax.numpy as jnp
from jax import lax
from jax.experimental import pallas as pl
from jax.experimental.pallas import tpu as pltpu

CTX = 200
PATH = 20
EMBED = 64
PATH_PAD = 32


def _body(ctx_idx_ref, path_idx_ref, bits_ref, ctx_table_ref, node_table_ref,
          o_ref, crows, nrows, sem):
    for i in range(CTX):
        pltpu.make_async_copy(
            ctx_table_ref.at[pl.ds(ctx_idx_ref[i], 1)],
            crows.at[pl.ds(i, 1)], sem).start()
    for i in range(PATH_PAD):
        pltpu.make_async_copy(
            node_table_ref.at[pl.ds(path_idx_ref[i], 1)],
            nrows.at[pl.ds(i, 1)], sem).start()

    def drain(i, _):
        pltpu.make_async_copy(
            node_table_ref.at[pl.ds(0, 1)], nrows.at[pl.ds(0, 1)], sem).wait()
        return 0

    lax.fori_loop(0, CTX + PATH_PAD, drain, 0)

    h = jnp.sum(crows[...], axis=0, keepdims=True) * (1.0 / CTX)
    n = nrows[...]
    b = bits_ref[...]
    t = jnp.sum(n * h, axis=1, keepdims=True)
    s = jax.nn.sigmoid(t)
    eps = 1e-9
    per = -b * jnp.log(s + eps) - (1.0 - b) * jnp.log(1.0 - s + eps)
    row = lax.broadcasted_iota(jnp.int32, (PATH_PAD, 1), 0)
    per = jnp.where(row < PATH, per, 0.0)
    o_ref[0, 0] = jnp.sum(per)


_call = pl.pallas_call(
    _body,
    in_specs=[
        pl.BlockSpec(memory_space=pltpu.SMEM),
        pl.BlockSpec(memory_space=pltpu.SMEM),
        pl.BlockSpec(memory_space=pltpu.VMEM),
        pl.BlockSpec(memory_space=pl.ANY),
        pl.BlockSpec(memory_space=pl.ANY),
    ],
    out_specs=pl.BlockSpec(memory_space=pltpu.SMEM),
    out_shape=jax.ShapeDtypeStruct((1, 1), jnp.float32),
    scratch_shapes=[
        pltpu.VMEM((CTX, EMBED), jnp.float32),
        pltpu.VMEM((PATH_PAD, EMBED), jnp.float32),
        pltpu.SemaphoreType.DMA,
    ],
)


def kernel(context_idx, path_indices, code_bits, context_table, node_table):
    ctx = jnp.asarray(context_idx, jnp.int32)
    pidx = jnp.asarray(path_indices, jnp.int32)
    path_pad = jnp.zeros((PATH_PAD,), jnp.int32).at[:PATH].set(pidx)
    bits_col = (jnp.zeros((PATH_PAD, 1), jnp.float32)
                .at[:PATH, 0].set(code_bits.astype(jnp.float32)))
    out = _call(ctx, path_pad, bits_col, context_table, node_table)
    return out[0, 0]


# speedup vs baseline: 1.0014x; 1.0014x over previous
import jax
import jax.numpy as jnp
from jax import lax
from jax.experimental import pallas as pl
from jax.experimental.pallas import tpu as pltpu

CTX = 200
PATH = 20
EMBED = 64
PATH_PAD = 32
NSEM = 8


def _body(ctx_idx_ref, path_idx_ref, bits_ref, ctx_table_ref, node_table_ref,
          o_ref, crows, nrows, sem):
    for i in range(CTX):
        pltpu.make_async_copy(
            ctx_table_ref.at[pl.ds(ctx_idx_ref[i], 1)],
            crows.at[pl.ds(i, 1)], sem.at[i % NSEM]).start()
    for i in range(PATH_PAD):
        pltpu.make_async_copy(
            node_table_ref.at[pl.ds(path_idx_ref[i], 1)],
            nrows.at[pl.ds(i, 1)], sem.at[(CTX + i) % NSEM]).start()

    for q in range(NSEM):
        cnt = (CTX + PATH_PAD + NSEM - 1 - q) // NSEM

        def drain(i, _, q=q):
            pltpu.make_async_copy(
                node_table_ref.at[pl.ds(0, 1)], nrows.at[pl.ds(0, 1)],
                sem.at[q]).wait()
            return 0

        lax.fori_loop(0, cnt, drain, 0)

    h = jnp.sum(crows[...], axis=0, keepdims=True) * (1.0 / CTX)
    n = nrows[...]
    b = bits_ref[...]
    t = jnp.sum(n * h, axis=1, keepdims=True)
    s = jax.nn.sigmoid(t)
    eps = 1e-9
    per = -b * jnp.log(s + eps) - (1.0 - b) * jnp.log(1.0 - s + eps)
    row = lax.broadcasted_iota(jnp.int32, (PATH_PAD, 1), 0)
    per = jnp.where(row < PATH, per, 0.0)
    o_ref[0, 0] = jnp.sum(per)


_call = pl.pallas_call(
    _body,
    in_specs=[
        pl.BlockSpec(memory_space=pltpu.SMEM),
        pl.BlockSpec(memory_space=pltpu.SMEM),
        pl.BlockSpec(memory_space=pltpu.VMEM),
        pl.BlockSpec(memory_space=pl.ANY),
        pl.BlockSpec(memory_space=pl.ANY),
    ],
    out_specs=pl.BlockSpec(memory_space=pltpu.SMEM),
    out_shape=jax.ShapeDtypeStruct((1, 1), jnp.float32),
    scratch_shapes=[
        pltpu.VMEM((CTX, EMBED), jnp.float32),
        pltpu.VMEM((PATH_PAD, EMBED), jnp.float32),
        pltpu.SemaphoreType.DMA((NSEM,)),
    ],
)


def kernel(context_idx, path_indices, code_bits, context_table, node_table):
    ctx = jnp.asarray(context_idx, jnp.int32)
    pidx = jnp.asarray(path_indices, jnp.int32)
    path_pad = jnp.zeros((PATH_PAD,), jnp.int32).at[:PATH].set(pidx)
    bits_col = (jnp.zeros((PATH_PAD, 1), jnp.float32)
                .at[:PATH, 0].set(code_bits.astype(jnp.float32)))
    out = _call(ctx, path_pad, bits_col, context_table, node_table)
    return out[0, 0]
